# baseline (speedup 1.0000x reference)
import jax
import jax.numpy as jnp
from jax import lax
from jax.experimental import pallas as pl

B, N, S = 4, 8192, 2048
D1, D2 = 16, 32
H1, H2 = 64, 64

NB1 = 512
NB2 = 2048


def _k1_body(dist_ref, p1_ref, p2_ref, w1a_ref, w1b_ref, b1_ref,
             y1_ref, sums_ref):
    d = dist_ref[0]
    p2 = p2_ref[0]

    p2hi = p2.astype(jnp.bfloat16)
    r1 = p2 - p2hi.astype(jnp.float32)
    p2mid = r1.astype(jnp.bfloat16)
    r2 = r1 - p2mid.astype(jnp.float32)
    p2lo = r2.astype(jnp.bfloat16)
    p2cat = jnp.concatenate([p2hi, p2mid, p2lo], axis=1)

    iota = lax.broadcasted_iota(jnp.int32, (1, S), 1)
    recips = []
    feats = []
    for _ in range(3):
        vmin = jnp.min(d, axis=1, keepdims=True)
        t = jnp.where(d == vmin, iota, S)
        imin = jnp.min(t, axis=1, keepdims=True)
        sel = (t == imin)
        g = lax.dot_general(sel.astype(jnp.bfloat16), p2cat,
                            (((1,), (0,)), ((), ())),
                            preferred_element_type=jnp.float32)
        feats.append((g[:, :D2] + g[:, D2:2 * D2]) + g[:, 2 * D2:])
        recips.append(1.0 / (vmin + 1e-8))
        d = jnp.where(sel, jnp.inf, d)
    norm = (recips[0] + recips[1]) + recips[2]
    interp = ((recips[0] / norm) * feats[0]
              + (recips[1] / norm) * feats[1]) + (recips[2] / norm) * feats[2]

    y1 = (lax.dot_general(p1_ref[0], w1a_ref[...], (((1,), (1,)), ((), ())),
                          preferred_element_type=jnp.float32)
          + lax.dot_general(interp, w1b_ref[...], (((1,), (1,)), ((), ())),
                            preferred_element_type=jnp.float32)
          + b1_ref[...])
    y1_ref[0] = y1

    s1 = jnp.sum(y1, axis=0, keepdims=True)
    s2 = jnp.sum(y1 * y1, axis=0, keepdims=True)
    sums_ref[0, 0] = jnp.concatenate(
        [s1, s2, jnp.zeros((6, H1), jnp.float32)], axis=0)


def _k2_body(y1_ref, sc_ref, sh_ref, w2_ref, b2_ref, y2_ref, sums_ref):
    h = jnp.maximum(y1_ref[0] * sc_ref[...] + sh_ref[...], 0.0)
    y2 = lax.dot_general(h, w2_ref[...], (((1,), (1,)), ((), ())),
                         preferred_element_type=jnp.float32) + b2_ref[...]
    y2_ref[0] = y2
    s1 = jnp.sum(y2, axis=0, keepdims=True)
    s2 = jnp.sum(y2 * y2, axis=0, keepdims=True)
    sums_ref[0, 0] = jnp.concatenate(
        [s1, s2, jnp.zeros((6, H2), jnp.float32)], axis=0)


def _k3_body(y2_ref, sc_ref, sh_ref, out_ref):
    out_ref[0] = jnp.maximum(y2_ref[0] * sc_ref[...] + sh_ref[...], 0.0)


@jax.jit
def kernel(xyz1, xyz2, points1, points2, W1, b1, g1, be1, W2, b2, g2, be2):
    nblk1 = N // NB1
    nblk2 = N // NB2

    dist = -2.0 * jnp.einsum('bnc,bmc->bnm', xyz1, xyz2)
    dist = dist + jnp.sum(xyz1 ** 2, axis=-1)[:, :, None]
    dist = dist + jnp.sum(xyz2 ** 2, axis=-1)[:, None, :]

    W1a = W1[:, :D1]
    W1b = W1[:, D1:]

    y1, sums1 = pl.pallas_call(
        _k1_body,
        grid=(B, nblk1),
        in_specs=[
            pl.BlockSpec((1, NB1, S), lambda b, i: (b, i, 0)),
            pl.BlockSpec((1, NB1, D1), lambda b, i: (b, i, 0)),
            pl.BlockSpec((1, S, D2), lambda b, i: (b, 0, 0)),
            pl.BlockSpec((H1, D1), lambda b, i: (0, 0)),
            pl.BlockSpec((H1, D2), lambda b, i: (0, 0)),
            pl.BlockSpec((1, H1), lambda b, i: (0, 0)),
        ],
        out_specs=[
            pl.BlockSpec((1, NB1, H1), lambda b, i: (b, i, 0)),
            pl.BlockSpec((1, 1, 8, H1), lambda b, i: (b, i, 0, 0)),
        ],
        out_shape=[
            jax.ShapeDtypeStruct((B, N, H1), jnp.float32),
            jax.ShapeDtypeStruct((B, nblk1, 8, H1), jnp.float32),
        ],
    )(dist, points1, points2, W1a, W1b, b1.reshape(1, H1))

    cnt = float(B * N)
    t = jnp.sum(sums1, axis=(0, 1))
    mean1, ex2 = t[0] / cnt, t[1] / cnt
    var1 = ex2 - mean1 * mean1
    sc1 = g1 / jnp.sqrt(var1 + 1e-5)
    sh1 = be1 - mean1 * sc1

    y2, sums2 = pl.pallas_call(
        _k2_body,
        grid=(B, nblk2),
        in_specs=[
            pl.BlockSpec((1, NB2, H1), lambda b, i: (b, i, 0)),
            pl.BlockSpec((1, H1), lambda b, i: (0, 0)),
            pl.BlockSpec((1, H1), lambda b, i: (0, 0)),
            pl.BlockSpec((H2, H1), lambda b, i: (0, 0)),
            pl.BlockSpec((1, H2), lambda b, i: (0, 0)),
        ],
        out_specs=[
            pl.BlockSpec((1, NB2, H2), lambda b, i: (b, i, 0)),
            pl.BlockSpec((1, 1, 8, H2), lambda b, i: (b, i, 0, 0)),
        ],
        out_shape=[
            jax.ShapeDtypeStruct((B, N, H2), jnp.float32),
            jax.ShapeDtypeStruct((B, nblk2, 8, H2), jnp.float32),
        ],
    )(y1, sc1.reshape(1, H1), sh1.reshape(1, H1), W2, b2.reshape(1, H2))

    t = jnp.sum(sums2, axis=(0, 1))
    mean2, ex2 = t[0] / cnt, t[1] / cnt
    var2 = ex2 - mean2 * mean2
    sc2 = g2 / jnp.sqrt(var2 + 1e-5)
    sh2 = be2 - mean2 * sc2

    out = pl.pallas_call(
        _k3_body,
        grid=(B, nblk2),
        in_specs=[
            pl.BlockSpec((1, NB2, H2), lambda b, i: (b, i, 0)),
            pl.BlockSpec((1, H2), lambda b, i: (0, 0)),
            pl.BlockSpec((1, H2), lambda b, i: (0, 0)),
        ],
        out_specs=pl.BlockSpec((1, NB2, H2), lambda b, i: (b, i, 0)),
        out_shape=jax.ShapeDtypeStruct((B, N, H2), jnp.float32),
    )(y2, sc2.reshape(1, H2), sh2.reshape(1, H2))

    return out


# speedup vs baseline: 1.1540x; 1.1540x over previous
import jax
import jax.numpy as jnp
from jax import lax
from jax.experimental import pallas as pl

B, N, S = 4, 8192, 2048
D1, D2 = 16, 32
H1, H2 = 64, 64

NB1 = 512
NB2 = 2048


def _k1_body(dist_ref, p1_ref, p2_ref, w1a_ref, w1b_ref, b1_ref,
             y1_ref, sums_ref):
    d = dist_ref[0]
    p2 = p2_ref[0]

    p2hi = p2.astype(jnp.bfloat16)
    r1 = p2 - p2hi.astype(jnp.float32)
    p2mid = r1.astype(jnp.bfloat16)
    r2 = r1 - p2mid.astype(jnp.float32)
    p2lo = r2.astype(jnp.bfloat16)
    p2cat = jnp.concatenate([p2hi, p2mid, p2lo], axis=1)

    iota = lax.broadcasted_iota(jnp.int32, (NB1, S), 1).astype(jnp.float32)
    recips = []
    feats = []
    for k in range(3):
        vmin = jnp.min(d, axis=1, keepdims=True)
        t = jnp.where(d == vmin, iota, float(S))
        imin = jnp.min(t, axis=1, keepdims=True)
        sel = (t == imin)
        g = lax.dot_general(sel.astype(jnp.bfloat16), p2cat,
                            (((1,), (0,)), ((), ())),
                            preferred_element_type=jnp.float32)
        feats.append((g[:, :D2] + g[:, D2:2 * D2]) + g[:, 2 * D2:])
        recips.append(1.0 / (vmin + 1e-8))
        if k < 2:
            d = jnp.where(sel, jnp.inf, d)
    norm = (recips[0] + recips[1]) + recips[2]
    interp = ((recips[0] / norm) * feats[0]
              + (recips[1] / norm) * feats[1]) + (recips[2] / norm) * feats[2]

    y1 = (lax.dot_general(p1_ref[0], w1a_ref[...], (((1,), (1,)), ((), ())),
                          preferred_element_type=jnp.float32)
          + lax.dot_general(interp, w1b_ref[...], (((1,), (1,)), ((), ())),
                            preferred_element_type=jnp.float32)
          + b1_ref[...])
    y1_ref[0] = y1

    s1 = jnp.sum(y1, axis=0, keepdims=True)
    s2 = jnp.sum(y1 * y1, axis=0, keepdims=True)
    sums_ref[0, 0] = jnp.concatenate(
        [s1, s2, jnp.zeros((6, H1), jnp.float32)], axis=0)


def _k2_body(y1_ref, sc_ref, sh_ref, w2_ref, b2_ref, y2_ref, sums_ref):
    h = jnp.maximum(y1_ref[0] * sc_ref[...] + sh_ref[...], 0.0)
    y2 = lax.dot_general(h, w2_ref[...], (((1,), (1,)), ((), ())),
                         preferred_element_type=jnp.float32) + b2_ref[...]
    y2_ref[0] = y2
    s1 = jnp.sum(y2, axis=0, keepdims=True)
    s2 = jnp.sum(y2 * y2, axis=0, keepdims=True)
    sums_ref[0, 0] = jnp.concatenate(
        [s1, s2, jnp.zeros((6, H2), jnp.float32)], axis=0)


def _k3_body(y2_ref, sc_ref, sh_ref, out_ref):
    out_ref[0] = jnp.maximum(y2_ref[0] * sc_ref[...] + sh_ref[...], 0.0)


@jax.jit
def kernel(xyz1, xyz2, points1, points2, W1, b1, g1, be1, W2, b2, g2, be2):
    nblk1 = N // NB1
    nblk2 = N // NB2

    dist = -2.0 * jnp.einsum('bnc,bmc->bnm', xyz1, xyz2)
    dist = dist + jnp.sum(xyz1 ** 2, axis=-1)[:, :, None]
    dist = dist + jnp.sum(xyz2 ** 2, axis=-1)[:, None, :]

    W1a = W1[:, :D1]
    W1b = W1[:, D1:]

    y1, sums1 = pl.pallas_call(
        _k1_body,
        grid=(B, nblk1),
        in_specs=[
            pl.BlockSpec((1, NB1, S), lambda b, i: (b, i, 0)),
            pl.BlockSpec((1, NB1, D1), lambda b, i: (b, i, 0)),
            pl.BlockSpec((1, S, D2), lambda b, i: (b, 0, 0)),
            pl.BlockSpec((H1, D1), lambda b, i: (0, 0)),
            pl.BlockSpec((H1, D2), lambda b, i: (0, 0)),
            pl.BlockSpec((1, H1), lambda b, i: (0, 0)),
        ],
        out_specs=[
            pl.BlockSpec((1, NB1, H1), lambda b, i: (b, i, 0)),
            pl.BlockSpec((1, 1, 8, H1), lambda b, i: (b, i, 0, 0)),
        ],
        out_shape=[
            jax.ShapeDtypeStruct((B, N, H1), jnp.float32),
            jax.ShapeDtypeStruct((B, nblk1, 8, H1), jnp.float32),
        ],
    )(dist, points1, points2, W1a, W1b, b1.reshape(1, H1))

    cnt = float(B * N)
    t = jnp.sum(sums1, axis=(0, 1))
    mean1, ex2 = t[0] / cnt, t[1] / cnt
    var1 = ex2 - mean1 * mean1
    sc1 = g1 / jnp.sqrt(var1 + 1e-5)
    sh1 = be1 - mean1 * sc1

    y2, sums2 = pl.pallas_call(
        _k2_body,
        grid=(B, nblk2),
        in_specs=[
            pl.BlockSpec((1, NB2, H1), lambda b, i: (b, i, 0)),
            pl.BlockSpec((1, H1), lambda b, i: (0, 0)),
            pl.BlockSpec((1, H1), lambda b, i: (0, 0)),
            pl.BlockSpec((H2, H1), lambda b, i: (0, 0)),
            pl.BlockSpec((1, H2), lambda b, i: (0, 0)),
        ],
        out_specs=[
            pl.BlockSpec((1, NB2, H2), lambda b, i: (b, i, 0)),
            pl.BlockSpec((1, 1, 8, H2), lambda b, i: (b, i, 0, 0)),
        ],
        out_shape=[
            jax.ShapeDtypeStruct((B, N, H2), jnp.float32),
            jax.ShapeDtypeStruct((B, nblk2, 8, H2), jnp.float32),
        ],
    )(y1, sc1.reshape(1, H1), sh1.reshape(1, H1), W2, b2.reshape(1, H2))

    t = jnp.sum(sums2, axis=(0, 1))
    mean2, ex2 = t[0] / cnt, t[1] / cnt
    var2 = ex2 - mean2 * mean2
    sc2 = g2 / jnp.sqrt(var2 + 1e-5)
    sh2 = be2 - mean2 * sc2

    out = pl.pallas_call(
        _k3_body,
        grid=(B, nblk2),
        in_specs=[
            pl.BlockSpec((1, NB2, H2), lambda b, i: (b, i, 0)),
            pl.BlockSpec((1, H2), lambda b, i: (0, 0)),
            pl.BlockSpec((1, H2), lambda b, i: (0, 0)),
        ],
        out_specs=pl.BlockSpec((1, NB2, H2), lambda b, i: (b, i, 0)),
        out_shape=jax.ShapeDtypeStruct((B, N, H2), jnp.float32),
    )(y2, sc2.reshape(1, H2), sh2.reshape(1, H2))

    return out
